# baseline (speedup 1.0000x reference)
import jax
import jax.numpy as jnp
from jax.experimental import pallas as pl
from jax.experimental.pallas import tpu as pltpu

N_CA, D_CA = 8, 16
N_SRO, D_SRO = 4, 32
D_RL = 64
DM = 128
N_TOK = N_CA + N_SRO + 1
OUTW = N_TOK * DM
D_TOT = N_CA * D_CA + N_SRO * D_SRO + D_RL

BLK = 1024


def _tok_kernel(idx_ref, wbd_ref, bias_ref, obs_ref, out_ref, wfull_ref):
    obs_dim = wfull_ref.shape[0]

    @pl.when(pl.program_id(0) == 0)
    def _build_wfull():
        iota = jax.lax.broadcasted_iota(jnp.int32, (obs_dim, D_TOT), 0)
        onehot = (iota == idx_ref[...]).astype(jnp.bfloat16)
        wfull_f32 = jax.lax.dot_general(
            onehot, wbd_ref[...],
            dimension_numbers=(((1,), (0,)), ((), ())),
            preferred_element_type=jnp.float32)
        wfull_ref[...] = wfull_f32.astype(jnp.bfloat16)

    obs_bf = obs_ref[...].astype(jnp.bfloat16)
    acc = jax.lax.dot_general(
        obs_bf, wfull_ref[...],
        dimension_numbers=(((1,), (0,)), ((), ())),
        preferred_element_type=jnp.float32)
    out_ref[...] = acc + bias_ref[...]


def kernel(obs, ca_idx, sro_idx, rl_idx, W_ca, b_ca, W_sro, b_sro, W_rl, b_rl, type_emb):
    batch, obs_dim = obs.shape

    idx = jnp.concatenate(
        [ca_idx.reshape(-1), sro_idx.reshape(-1), rl_idx.reshape(-1)]
    ).astype(jnp.int32)[None, :]

    wbd = jnp.zeros((D_TOT, N_TOK, DM), jnp.float32)
    for t in range(N_CA):
        wbd = wbd.at[t * D_CA:(t + 1) * D_CA, t, :].set(W_ca)
    base = N_CA * D_CA
    for t in range(N_SRO):
        wbd = wbd.at[base + t * D_SRO:base + (t + 1) * D_SRO, N_CA + t, :].set(W_sro)
    wbd = wbd.at[base + N_SRO * D_SRO:, N_TOK - 1, :].set(W_rl)
    wbd = wbd.reshape(D_TOT, OUTW).astype(jnp.bfloat16)

    btok = jnp.concatenate([
        jnp.broadcast_to(b_ca, (N_CA, DM)),
        jnp.broadcast_to(b_sro, (N_SRO, DM)),
        b_rl[None, :],
    ], axis=0)
    bias = (type_emb + btok).reshape(1, OUTW)

    grid = (batch // BLK,)
    out = pl.pallas_call(
        _tok_kernel,
        grid=grid,
        in_specs=[
            pl.BlockSpec((1, D_TOT), lambda i: (0, 0)),
            pl.BlockSpec((D_TOT, OUTW), lambda i: (0, 0)),
            pl.BlockSpec((1, OUTW), lambda i: (0, 0)),
            pl.BlockSpec((BLK, obs_dim), lambda i: (i, 0)),
        ],
        out_specs=pl.BlockSpec((BLK, OUTW), lambda i: (i, 0)),
        out_shape=jax.ShapeDtypeStruct((batch, OUTW), jnp.float32),
        scratch_shapes=[pltpu.VMEM((obs_dim, OUTW), jnp.bfloat16)],
        compiler_params=pltpu.CompilerParams(
            dimension_semantics=("arbitrary",)),
    )(idx, wbd, bias, obs)
    return out.reshape(batch, N_TOK, DM)


# speedup vs baseline: 1.4639x; 1.4639x over previous
import jax
import jax.numpy as jnp
from jax.experimental import pallas as pl
from jax.experimental.pallas import tpu as pltpu

N_CA, D_CA = 8, 16
N_SRO, D_SRO = 4, 32
D_RL = 64
DM = 128
N_TOK = N_CA + N_SRO + 1
OUTW = N_TOK * DM
D_TOT = N_CA * D_CA + N_SRO * D_SRO + D_RL

BLK = 1024


def _build_wfull_kernel(idx_ref, wbd_ref, wfull_ref):
    obs_dim = wfull_ref.shape[0]
    iota = jax.lax.broadcasted_iota(jnp.int32, (obs_dim, D_TOT), 0)
    onehot = (iota == idx_ref[...]).astype(jnp.bfloat16)
    wfull_f32 = jax.lax.dot_general(
        onehot, wbd_ref[...],
        dimension_numbers=(((1,), (0,)), ((), ())),
        preferred_element_type=jnp.float32)
    wfull_ref[...] = wfull_f32.astype(jnp.bfloat16)


def _tok_kernel(wfull_ref, bias_ref, obs_ref, out_ref):
    obs_bf = obs_ref[...].astype(jnp.bfloat16)
    acc = jax.lax.dot_general(
        obs_bf, wfull_ref[...],
        dimension_numbers=(((1,), (0,)), ((), ())),
        preferred_element_type=jnp.float32)
    acc = acc + bias_ref[...]
    for t in range(N_TOK):
        out_ref[:, t, :] = acc[:, t * DM:(t + 1) * DM]


def kernel(obs, ca_idx, sro_idx, rl_idx, W_ca, b_ca, W_sro, b_sro, W_rl, b_rl, type_emb):
    batch, obs_dim = obs.shape

    idx = jnp.concatenate(
        [ca_idx.reshape(-1), sro_idx.reshape(-1), rl_idx.reshape(-1)]
    ).astype(jnp.int32)[None, :]

    wbd = jnp.zeros((D_TOT, N_TOK, DM), jnp.float32)
    for t in range(N_CA):
        wbd = wbd.at[t * D_CA:(t + 1) * D_CA, t, :].set(W_ca)
    base = N_CA * D_CA
    for t in range(N_SRO):
        wbd = wbd.at[base + t * D_SRO:base + (t + 1) * D_SRO, N_CA + t, :].set(W_sro)
    wbd = wbd.at[base + N_SRO * D_SRO:, N_TOK - 1, :].set(W_rl)
    wbd = wbd.reshape(D_TOT, OUTW).astype(jnp.bfloat16)

    btok = jnp.concatenate([
        jnp.broadcast_to(b_ca, (N_CA, DM)),
        jnp.broadcast_to(b_sro, (N_SRO, DM)),
        b_rl[None, :],
    ], axis=0)
    bias = (type_emb + btok).reshape(1, OUTW)

    wfull = pl.pallas_call(
        _build_wfull_kernel,
        out_shape=jax.ShapeDtypeStruct((obs_dim, OUTW), jnp.bfloat16),
    )(idx, wbd)

    grid = (batch // BLK,)
    out = pl.pallas_call(
        _tok_kernel,
        grid=grid,
        in_specs=[
            pl.BlockSpec((obs_dim, OUTW), lambda i: (0, 0)),
            pl.BlockSpec((1, OUTW), lambda i: (0, 0)),
            pl.BlockSpec((BLK, obs_dim), lambda i: (i, 0)),
        ],
        out_specs=pl.BlockSpec((BLK, N_TOK, DM), lambda i: (i, 0, 0)),
        out_shape=jax.ShapeDtypeStruct((batch, N_TOK, DM), jnp.float32),
        compiler_params=pltpu.CompilerParams(
            dimension_semantics=("parallel",)),
    )(wfull, bias, obs)
    return out


# speedup vs baseline: 4.0574x; 2.7716x over previous
import jax
import jax.numpy as jnp
from jax.experimental import pallas as pl
from jax.experimental.pallas import tpu as pltpu

N_CA, D_CA = 8, 16
N_SRO, D_SRO = 4, 32
D_RL = 64
DM = 128
N_TOK = N_CA + N_SRO + 1
OUTW = N_TOK * DM
D_TOT = N_CA * D_CA + N_SRO * D_SRO + D_RL

BLK = 1024


def _build_wfull_kernel(idx_ref, wbd_ref, wfull_ref):
    obs_dim = wfull_ref.shape[0]
    iota = jax.lax.broadcasted_iota(jnp.int32, (obs_dim, D_TOT), 0)
    onehot = (iota == idx_ref[...]).astype(jnp.bfloat16)
    wfull_f32 = jax.lax.dot_general(
        onehot, wbd_ref[...],
        dimension_numbers=(((1,), (0,)), ((), ())),
        preferred_element_type=jnp.float32)
    wfull_ref[...] = wfull_f32.astype(jnp.bfloat16)


def _tok_kernel(wfull_ref, bias_ref, obs_ref, out_ref):
    obs_bf = obs_ref[...].astype(jnp.bfloat16)
    acc = jax.lax.dot_general(
        obs_bf, wfull_ref[...],
        dimension_numbers=(((1,), (0,)), ((), ())),
        preferred_element_type=jnp.float32)
    acc = acc + bias_ref[...]
    for t in range(N_TOK):
        out_ref[t, :, :] = acc[:, t * DM:(t + 1) * DM]


def kernel(obs, ca_idx, sro_idx, rl_idx, W_ca, b_ca, W_sro, b_sro, W_rl, b_rl, type_emb):
    batch, obs_dim = obs.shape

    idx = jnp.concatenate(
        [ca_idx.reshape(-1), sro_idx.reshape(-1), rl_idx.reshape(-1)]
    ).astype(jnp.int32)[None, :]

    wbd = jnp.zeros((D_TOT, N_TOK, DM), jnp.float32)
    for t in range(N_CA):
        wbd = wbd.at[t * D_CA:(t + 1) * D_CA, t, :].set(W_ca)
    base = N_CA * D_CA
    for t in range(N_SRO):
        wbd = wbd.at[base + t * D_SRO:base + (t + 1) * D_SRO, N_CA + t, :].set(W_sro)
    wbd = wbd.at[base + N_SRO * D_SRO:, N_TOK - 1, :].set(W_rl)
    wbd = wbd.reshape(D_TOT, OUTW).astype(jnp.bfloat16)

    btok = jnp.concatenate([
        jnp.broadcast_to(b_ca, (N_CA, DM)),
        jnp.broadcast_to(b_sro, (N_SRO, DM)),
        b_rl[None, :],
    ], axis=0)
    bias = (type_emb + btok).reshape(1, OUTW)

    wfull = pl.pallas_call(
        _build_wfull_kernel,
        out_shape=jax.ShapeDtypeStruct((obs_dim, OUTW), jnp.bfloat16),
    )(idx, wbd)

    grid = (batch // BLK,)
    out = pl.pallas_call(
        _tok_kernel,
        grid=grid,
        in_specs=[
            pl.BlockSpec((obs_dim, OUTW), lambda i: (0, 0)),
            pl.BlockSpec((1, OUTW), lambda i: (0, 0)),
            pl.BlockSpec((BLK, obs_dim), lambda i: (i, 0)),
        ],
        out_specs=pl.BlockSpec((N_TOK, BLK, DM), lambda i: (0, i, 0)),
        out_shape=jax.ShapeDtypeStruct((N_TOK, batch, DM), jnp.float32),
        compiler_params=pltpu.CompilerParams(
            dimension_semantics=("parallel",)),
    )(wfull, bias, obs)
    return jnp.transpose(out, (1, 0, 2))
